# baseline (speedup 1.0000x reference)
import functools

import jax
import jax.numpy as jnp
from jax import lax
from jax.experimental import pallas as pl
from jax.experimental.pallas import tpu as pltpu
from jax.experimental.pallas import tpu_sc as plsc

_B, _L, _D = 4096, 200, 64
_HALF = _L // 2
_NC, _NS = 2, 16
_NW = _NC * _NS
_BPW = _B // _NW
_CH = _D // 16
_IPW = _BPW * _L
_H0, _H1 = 104, 96
_TRC = 32768
_TRS = _TRC.bit_length() - 1
_TRM = _TRC // 4 - 1


def _tr_body(tin_ref, out_ref):
    eye = jnp.eye(_D, dtype=jnp.float32)
    t = jax.lax.dot_general(tin_ref[...], eye, (((0,), (0,)), ((), ())),
                            preferred_element_type=jnp.float32)
    r = jax.lax.bitcast_convert_type(t, jnp.uint32) + jnp.uint32(0x8000)
    lo = r[:, 0:_D // 2] >> 16
    hi = r[:, _D // 2:_D] & jnp.uint32(0xFFFF0000)
    packed = jax.lax.bitcast_convert_type(hi | lo, jnp.int32)
    q4 = _TRC // 4
    for q in range(4):
        out_ref[:, 32 * q:32 * q + 32] = packed[q * q4:(q + 1) * q4]


def _pack_pairs(table_t):
    v = table_t.shape[1]
    grid = (v + _TRC - 1) // _TRC
    return pl.pallas_call(
        _tr_body,
        grid=(grid,),
        in_specs=[pl.BlockSpec((_D, _TRC), lambda k: (0, k))],
        out_specs=pl.BlockSpec((_TRC // 4, 2 * _D), lambda k: (k, 0)),
        out_shape=jax.ShapeDtypeStruct((grid * (_TRC // 4), 2 * _D),
                                       jnp.int32),
    )(table_t)


def _bag_body(idx_hbm, table_hbm, out_hbm,
              idx_v, idxp_v, rows_v, acc_v, sem0, sem1):
    wid = lax.axis_index("s") * _NC + lax.axis_index("c")
    base = wid * _BPW
    pltpu.sync_copy(idx_hbm.at[pl.ds(base * _L, _IPW)],
                    idx_v.at[pl.ds(0, _IPW)])
    sems = (sem0, sem1)
    starts = (0, _H0)
    sizes = (_H0, _H1)

    def prep_issue(bag, h, buf):
        off = bag * _L + starts[h]
        n = sizes[h]
        for t in range(-(-n // 16)):
            v = idx_v[pl.ds(off + 16 * t, 16)]
            idxp_v[buf, pl.ds(16 * t, 16)] = (
                ((v >> _TRS) << (_TRS - 2)) | (v & _TRM))
        pltpu.async_copy(table_hbm.at[idxp_v.at[buf, pl.ds(0, n)]],
                         rows_v.at[buf, pl.ds(0, n)], sems[buf])

    def wait(buf, h):
        n = sizes[h]
        pltpu.make_async_copy(table_hbm.at[pl.ds(0, n)],
                              rows_v.at[buf, pl.ds(0, n)], sems[buf]).wait()

    def accumulate(buf, bag, h, accs):
        off = bag * _L + starts[h]
        n = sizes[h]

        def do16(r0, lo, accs):
            ov = ((idx_v[pl.ds(off + r0, 16)] >> (_TRS - 2)) & 3) << 5
            for j in range(lo, 16):
                o = pl.multiple_of(ov[j], 32)
                a0, a1, a2, a3 = accs
                v0 = rows_v[buf, r0 + j, pl.ds(o, 16)]
                v1 = rows_v[buf, r0 + j, pl.ds(o + 16, 16)]
                b0 = plsc.bitcast(v0, jnp.bfloat16)
                b1 = plsc.bitcast(v1, jnp.bfloat16)
                c0, c2 = plsc.unpack(b0, format=plsc.PackFormat.INTERLEAVED)
                c1, c3 = plsc.unpack(b1, format=plsc.PackFormat.INTERLEAVED)
                accs = (a0 + c0, a1 + c1, a2 + c2, a3 + c3)
            return accs

        def gbody(g, accs):
            return do16(16 * g, 0, accs)

        ngrp = n // 16
        accs = lax.fori_loop(0, ngrp, gbody, accs)
        if n % 16:
            accs = do16(n - 16, 16 - (n - ngrp * 16), accs)
        return accs

    prep_issue(0, 0, 0)

    def bag_body(bag, carry):
        prep_issue(bag, 1, 1)
        wait(0, 0)
        accs = tuple(jnp.zeros((16,), jnp.float32) for _ in range(_CH))
        accs = accumulate(0, bag, 0, accs)

        @pl.when(bag + 1 < _BPW)
        def _():
            prep_issue(bag + 1, 0, 0)

        wait(1, 1)
        accs = accumulate(1, bag, 1, accs)
        for c in range(_CH):
            acc_v[bag, pl.ds(16 * c, 16)] = accs[c]
        return carry

    lax.fori_loop(0, _BPW, bag_body, 0)
    pltpu.sync_copy(acc_v, out_hbm.at[pl.ds(base, _BPW)])


_bag_sum = functools.partial(
    pl.kernel,
    out_type=jax.ShapeDtypeStruct((_B, _D), jnp.float32),
    mesh=plsc.VectorSubcoreMesh(core_axis_name="c", subcore_axis_name="s"),
    scratch_types=[
        pltpu.VMEM((_IPW + 16,), jnp.int32),
        pltpu.VMEM((2, 112), jnp.int32),
        pltpu.VMEM((2, _H0, 2 * _D), jnp.int32),
        pltpu.VMEM((_BPW, _D), jnp.float32),
        pltpu.SemaphoreType.DMA,
        pltpu.SemaphoreType.DMA,
    ],
    compiler_params=pltpu.CompilerParams(needs_layout_passes=False),
)(_bag_body)


def _mlp_body(bag_ref, w1_ref, b1_ref, w2_ref, b2_ref, out_ref):
    emb = bag_ref[...] * (1.0 / _L)
    h = jnp.dot(emb, w1_ref[...], preferred_element_type=jnp.float32)
    h = jnp.maximum(h + b1_ref[...], 0.0)
    logits = jnp.dot(h, w2_ref[...], preferred_element_type=jnp.float32)
    logits = logits + b2_ref[...]
    m = jnp.max(logits, axis=1, keepdims=True)
    e = jnp.exp(logits - m)
    out_ref[...] = e / jnp.sum(e, axis=1, keepdims=True)


def kernel(x, table, W1, b1, W2, b2):
    nc = W2.shape[1]
    xi = x.astype(jnp.int32).reshape(_B * _L)
    tbl2 = _pack_pairs(table.T)
    bag = _bag_sum(xi, tbl2)
    return pl.pallas_call(
        _mlp_body,
        out_shape=jax.ShapeDtypeStruct((_B, nc), jnp.float32),
    )(bag, W1, b1.reshape(1, _D), W2, b2.reshape(1, nc))


# speedup vs baseline: 1.3239x; 1.3239x over previous
import functools

import jax
import jax.numpy as jnp
from jax import lax
from jax.experimental import pallas as pl
from jax.experimental.pallas import tpu as pltpu
from jax.experimental.pallas import tpu_sc as plsc

_B, _L, _D = 4096, 200, 64
_HALF = _L // 2
_NC, _NS = 2, 16
_NW = _NC * _NS
_BPW = _B // _NW
_CH = _D // 16
_IPW = _BPW * _L
_H0, _H1 = 104, 96
_TRC = 32768
_TRS = _TRC.bit_length() - 1
_TRM = _TRC // 2 - 1


def _tr_body(tin_ref, out_ref):
    eye = jnp.eye(_D, dtype=jnp.float32)
    t = jax.lax.dot_general(tin_ref[...], eye, (((0,), (0,)), ((), ())),
                            preferred_element_type=jnp.float32)
    out_ref[:, 0:_D] = t[0:_TRC // 2]
    out_ref[:, _D:2 * _D] = t[_TRC // 2:_TRC]


def _pack_pairs(table_t):
    v = table_t.shape[1]
    grid = (v + _TRC - 1) // _TRC
    return pl.pallas_call(
        _tr_body,
        grid=(grid,),
        in_specs=[pl.BlockSpec((_D, _TRC), lambda k: (0, k))],
        out_specs=pl.BlockSpec((_TRC // 2, 2 * _D), lambda k: (k, 0)),
        out_shape=jax.ShapeDtypeStruct((grid * (_TRC // 2), 2 * _D),
                                       jnp.float32),
    )(table_t)


def _bag_body(idx_hbm, table_hbm, out_hbm,
              idx_v, idxp_v, rows_v, acc_v, sem0, sem1):
    wid = lax.axis_index("s") * _NC + lax.axis_index("c")
    base = wid * _BPW
    pltpu.sync_copy(idx_hbm.at[pl.ds(base * _L, _IPW)],
                    idx_v.at[pl.ds(0, _IPW)])
    sems = (sem0, sem1)
    starts = (0, _H0)
    sizes = (_H0, _H1)

    def prep_issue(bag, h, buf):
        off = bag * _L + starts[h]
        n = sizes[h]
        for t in range(-(-n // 16)):
            v = idx_v[pl.ds(off + 16 * t, 16)]
            idxp_v[buf, pl.ds(16 * t, 16)] = (
                ((v >> _TRS) << (_TRS - 1)) | (v & _TRM))
        pltpu.async_copy(table_hbm.at[idxp_v.at[buf, pl.ds(0, n)]],
                         rows_v.at[buf, pl.ds(0, n)], sems[buf])

    def wait(buf, h):
        n = sizes[h]
        pltpu.make_async_copy(table_hbm.at[pl.ds(0, n)],
                              rows_v.at[buf, pl.ds(0, n)], sems[buf]).wait()

    def accumulate(buf, bag, h, accs):
        off = bag * _L + starts[h]
        n = sizes[h]

        def do16(r0, lo, accs):
            ov = ((idx_v[pl.ds(off + r0, 16)] >> (_TRS - 1)) & 1) << 6
            for j in range(lo, 16):
                o = ov[j]
                accs = tuple(accs[c] + rows_v[buf, r0 + j,
                                              pl.ds(o + 16 * c, 16)]
                             for c in range(_CH))
            return accs

        def gbody(g, accs):
            return do16(16 * g, 0, accs)

        ngrp = n // 16
        accs = lax.fori_loop(0, ngrp, gbody, accs)
        if n % 16:
            accs = do16(n - 16, 16 - (n - ngrp * 16), accs)
        return accs

    prep_issue(0, 0, 0)

    def bag_body(bag, carry):
        prep_issue(bag, 1, 1)
        wait(0, 0)
        accs = tuple(jnp.zeros((16,), jnp.float32) for _ in range(_CH))
        accs = accumulate(0, bag, 0, accs)

        @pl.when(bag + 1 < _BPW)
        def _():
            prep_issue(bag + 1, 0, 0)

        wait(1, 1)
        accs = accumulate(1, bag, 1, accs)
        for c in range(_CH):
            acc_v[bag, pl.ds(16 * c, 16)] = accs[c]
        return carry

    lax.fori_loop(0, _BPW, bag_body, 0)
    pltpu.sync_copy(acc_v, out_hbm.at[pl.ds(base, _BPW)])


_bag_sum = functools.partial(
    pl.kernel,
    out_type=jax.ShapeDtypeStruct((_B, _D), jnp.float32),
    mesh=plsc.VectorSubcoreMesh(core_axis_name="c", subcore_axis_name="s"),
    scratch_types=[
        pltpu.VMEM((_IPW + 16,), jnp.int32),
        pltpu.VMEM((2, 112), jnp.int32),
        pltpu.VMEM((2, _H0, 2 * _D), jnp.float32),
        pltpu.VMEM((_BPW, _D), jnp.float32),
        pltpu.SemaphoreType.DMA,
        pltpu.SemaphoreType.DMA,
    ],
)(_bag_body)


def _mlp_body(bag_ref, w1_ref, b1_ref, w2_ref, b2_ref, out_ref):
    emb = bag_ref[...] * (1.0 / _L)
    h = jnp.dot(emb, w1_ref[...], preferred_element_type=jnp.float32)
    h = jnp.maximum(h + b1_ref[...], 0.0)
    logits = jnp.dot(h, w2_ref[...], preferred_element_type=jnp.float32)
    logits = logits + b2_ref[...]
    m = jnp.max(logits, axis=1, keepdims=True)
    e = jnp.exp(logits - m)
    out_ref[...] = e / jnp.sum(e, axis=1, keepdims=True)


def kernel(x, table, W1, b1, W2, b2):
    nc = W2.shape[1]
    xi = x.astype(jnp.int32).reshape(_B * _L)
    tbl2 = _pack_pairs(table.T)
    bag = _bag_sum(xi, tbl2)
    return pl.pallas_call(
        _mlp_body,
        out_shape=jax.ShapeDtypeStruct((_B, nc), jnp.float32),
    )(bag, W1, b1.reshape(1, _D), W2, b2.reshape(1, nc))
